# baseline (speedup 1.0000x reference)
import functools

import jax
import jax.numpy as jnp
from jax import lax
from jax.experimental import pallas as pl
from jax.experimental.pallas import tpu as pltpu
from jax.experimental.pallas import tpu_sc as plsc

HIDDEN_DIM = 768
NUM_EXPERTS = 8
TOP_K = 2
N_TOKENS = 32768

BLOCK = 4096
N_STEPS = N_TOKENS // BLOCK

N_SPLIT = 4
SUB = BLOCK // N_SPLIT

SC_ROWS = 4096
NW = 32
ROWS_PER_W = SC_ROWS // NW


def _sc_copy_kernel(x_hbm, out_hbm, buf, sem):
    wid = lax.axis_index("c") * 16 + lax.axis_index("s")
    base = wid * ROWS_PER_W
    cp = pltpu.make_async_copy(
        x_hbm.at[pl.ds(base, ROWS_PER_W)], buf, sem)
    cp.start()
    cp.wait()
    cp2 = pltpu.make_async_copy(
        buf, out_hbm.at[pl.ds(base, ROWS_PER_W)], sem)
    cp2.start()
    cp2.wait()


def _sc_copy(x):
    mesh = plsc.VectorSubcoreMesh(core_axis_name="c", subcore_axis_name="s")
    return pl.kernel(
        _sc_copy_kernel,
        mesh=mesh,
        out_type=jax.ShapeDtypeStruct((SC_ROWS, HIDDEN_DIM), jnp.float32),
        scratch_types=[
            pltpu.VMEM((ROWS_PER_W, HIDDEN_DIM), jnp.float32),
            pltpu.SemaphoreType.DMA,
        ],
    )(x[:SC_ROWS])


def _start_block_copy(x_hbm, xbuf, xsem, step, slot):
    for q in range(N_SPLIT):
        pltpu.make_async_copy(
            x_hbm.at[pl.ds(step * BLOCK + q * SUB, SUB)],
            xbuf.at[slot, pl.ds(q * SUB, SUB)],
            xsem.at[slot, q]).start()


def _wait_block_copy(x_hbm, xbuf, xsem, step, slot):
    for q in range(N_SPLIT):
        pltpu.make_async_copy(
            x_hbm.at[pl.ds(step * BLOCK + q * SUB, SUB)],
            xbuf.at[slot, pl.ds(q * SUB, SUB)],
            xsem.at[slot, q]).wait()


def _router_kernel(x_hbm, hp_hbm, idxt_ref, xbuf, hpbuf, xsem, hpsem):
    i = pl.program_id(0)

    @pl.when(i == 0)
    def _():
        _start_block_copy(x_hbm, xbuf, xsem, 0, 0)
        hp_cp = pltpu.make_async_copy(hp_hbm, hpbuf, hpsem)
        hp_cp.start()
        hp_cp.wait()

    @pl.when(i + 1 < N_STEPS)
    def _():
        _start_block_copy(x_hbm, xbuf, xsem, i + 1, (i + 1) % 2)

    _wait_block_copy(x_hbm, xbuf, xsem, i, i % 2)

    x = xbuf[i % 2]
    hp = hpbuf[...]
    scores = jnp.abs(
        jax.lax.dot_general(
            hp, x, (((1,), (1,)), ((), ())),
            preferred_element_type=jnp.float32,
        )
    )
    iota = jax.lax.broadcasted_iota(jnp.int32, scores.shape, 0)
    m1 = jnp.max(scores, axis=0, keepdims=True)
    i1 = jnp.min(jnp.where(scores == m1, iota, NUM_EXPERTS),
                 axis=0, keepdims=True)
    masked = jnp.where(iota == i1, -1.0, scores)
    m2 = jnp.max(masked, axis=0, keepdims=True)
    i2 = jnp.min(jnp.where(masked == m2, iota, NUM_EXPERTS),
                 axis=0, keepdims=True)
    idxt_ref[...] = jnp.concatenate([i1, i2], axis=0)


def kernel(x, hash_planes):
    n = x.shape[0]
    dummy = _sc_copy(x)
    idxt = pl.pallas_call(
        _router_kernel,
        grid=(N_STEPS,),
        in_specs=[
            pl.BlockSpec(memory_space=pltpu.MemorySpace.HBM),
            pl.BlockSpec(memory_space=pltpu.MemorySpace.HBM),
        ],
        out_specs=pl.BlockSpec((TOP_K, BLOCK), lambda i: (0, i)),
        out_shape=jax.ShapeDtypeStruct((TOP_K, n), jnp.int32),
        scratch_shapes=[
            pltpu.VMEM((2, BLOCK, HIDDEN_DIM), jnp.float32),
            pltpu.VMEM((NUM_EXPERTS, HIDDEN_DIM), jnp.float32),
            pltpu.SemaphoreType.DMA((2, N_SPLIT)),
            pltpu.SemaphoreType.DMA,
        ],
        compiler_params=pltpu.CompilerParams(
            dimension_semantics=("arbitrary",),
        ),
    )(x, hash_planes)
    topk_indices = idxt.T
    topk_probs = jnp.full((n, TOP_K), 1.0 / TOP_K, jnp.float32) + 0.0 * dummy[0, 0]
    probs_uniform = jnp.full((n, NUM_EXPERTS), 1.0 / NUM_EXPERTS, jnp.float32)
    return (topk_indices, topk_probs, probs_uniform)
